# Initial kernel scaffold; baseline (speedup 1.0000x reference)
import jax
import jax.numpy as jnp
from jax.experimental import pallas as pl


def kernel(E, R, E_t, R_ht, R_ht_width, R_tt, R_tt_width, s, r, o):
    raise NotImplementedError("write your pallas kernel here")


# speedup vs baseline: 1.2305x; 1.2305x over previous
import functools

import jax
import jax.numpy as jnp
from jax import lax
from jax.experimental import pallas as pl
from jax.experimental.pallas import tpu as pltpu
from jax.experimental.pallas import tpu_sc as plsc

DIM = 64
PSI = 2.0
MULT = 20.0
L = 16
NCHUNK_DIM = DIM // L

_info = plsc.get_sparse_core_info()
NC = _info.num_cores
NS = _info.num_subcores
NW = NC * NS


def _body(e_hbm, et_hbm, rcat_hbm, s_hbm, r_hbm, o_hbm, out_hbm,
          sidx, ridx, oidx, es_v, eo_v, ets_v, eto_v, rc_v, out_v, sem):
    rows_per_w = s_hbm.shape[0] // NW
    C = es_v.shape[0]
    n_chunks = rows_per_w // C
    wid = lax.axis_index("s") * NC + lax.axis_index("c")
    base = wid * rows_per_w

    pltpu.sync_copy(s_hbm.at[pl.ds(base, rows_per_w)], sidx)
    pltpu.sync_copy(r_hbm.at[pl.ds(base, rows_per_w)], ridx)
    pltpu.sync_copy(o_hbm.at[pl.ds(base, rows_per_w)], oidx)

    lane = lax.iota(jnp.int32, L)

    @pl.loop(0, n_chunks)
    def _chunk(c):
        off = c * C
        i_s = sidx.at[pl.ds(off, C)]
        i_r = ridx.at[pl.ds(off, C)]
        i_o = oidx.at[pl.ds(off, C)]
        cps = [
            pltpu.async_copy(e_hbm.at[i_s], es_v, sem),
            pltpu.async_copy(e_hbm.at[i_o], eo_v, sem),
            pltpu.async_copy(et_hbm.at[i_s], ets_v, sem),
            pltpu.async_copy(et_hbm.at[i_o], eto_v, sem),
            pltpu.async_copy(rcat_hbm.at[i_r], rc_v, sem),
        ]
        for cp in cps:
            cp.wait()

        @pl.loop(0, C // L)
        def _group(g):
            rows = g * L + lane

            def dstep(d, carry):
                (acc_b, dmx_h, pp_h, plo_h, phi_h,
                 dmx_t, pp_t, plo_t, phi_t) = carry
                dv = jnp.full((L,), 0, jnp.int32) + d
                es = plsc.load_gather(es_v, [rows, dv])
                rr = plsc.load_gather(rc_v, [rows, dv])
                eo = plsc.load_gather(eo_v, [rows, dv])
                acc_b = acc_b + es * rr * eo
                p = plsc.load_gather(ets_v, [rows, dv])
                lo = plsc.load_gather(rc_v, [rows, dv + DIM])
                w = plsc.load_gather(rc_v, [rows, dv + 2 * DIM])
                hi = lo + jnp.maximum(w, 0.0)
                m = jnp.maximum(p - hi, jnp.maximum(lo - p, 0.0))
                dmx_h = jnp.maximum(dmx_h, m)
                pp_h = pp_h + p * p
                plo_h = plo_h + p * lo
                phi_h = phi_h + p * hi
                p = plsc.load_gather(eto_v, [rows, dv])
                lo = plsc.load_gather(rc_v, [rows, dv + 3 * DIM])
                w = plsc.load_gather(rc_v, [rows, dv + 4 * DIM])
                hi = lo + jnp.maximum(w, 0.0)
                m = jnp.maximum(p - hi, jnp.maximum(lo - p, 0.0))
                dmx_t = jnp.maximum(dmx_t, m)
                pp_t = pp_t + p * p
                plo_t = plo_t + p * lo
                phi_t = phi_t + p * hi
                return (acc_b, dmx_h, pp_h, plo_h, phi_h,
                        dmx_t, pp_t, plo_t, phi_t)

            z = jnp.zeros((L,), jnp.float32)
            (acc_b, dmx_h, pp_h, plo_h, phi_h,
             dmx_t, pp_t, plo_t, phi_t) = lax.fori_loop(
                 0, DIM, dstep, (z, z, z, z, z, z, z, z, z), unroll=4)
            dh = jnp.where(dmx_h > 0.0, jnp.maximum(plo_h, phi_h), pp_h)
            dt = jnp.where(dmx_t > 0.0, jnp.maximum(plo_t, phi_t), pp_t)
            denom = ((1.0 + jnp.exp(-PSI * acc_b))
                     * (1.0 + jnp.exp(PSI * dh))
                     * (1.0 + jnp.exp(PSI * dt)))
            out_v[pl.ds(g * L, L)] = MULT / denom

        pltpu.sync_copy(out_v, out_hbm.at[pl.ds(base + off, C)])


@functools.partial(jax.jit, static_argnames=("chunk",))
def _score(e, et, rcat, s, r, o, chunk=64):
    B = s.shape[0]
    rows_per_w = B // NW
    mesh = plsc.VectorSubcoreMesh(core_axis_name="c", subcore_axis_name="s")
    return pl.kernel(
        _body,
        out_type=jax.ShapeDtypeStruct((B,), jnp.float32),
        mesh=mesh,
        compiler_params=pltpu.CompilerParams(needs_layout_passes=False,
                                             use_tc_tiling_on_sc=False),
        scratch_types=[
            pltpu.VMEM((rows_per_w,), jnp.int32),
            pltpu.VMEM((rows_per_w,), jnp.int32),
            pltpu.VMEM((rows_per_w,), jnp.int32),
            pltpu.VMEM((chunk, DIM), jnp.float32),
            pltpu.VMEM((chunk, DIM), jnp.float32),
            pltpu.VMEM((chunk, DIM), jnp.float32),
            pltpu.VMEM((chunk, DIM), jnp.float32),
            pltpu.VMEM((chunk, 5 * DIM), jnp.float32),
            pltpu.VMEM((chunk,), jnp.float32),
            pltpu.SemaphoreType.DMA,
        ],
    )(e, et, rcat, s, r, o)


def kernel(E, R, E_t, R_ht, R_ht_width, R_tt, R_tt_width, s, r, o):
    rcat = jnp.concatenate([R, R_ht, R_ht_width, R_tt, R_tt_width], axis=1)
    return _score(E, E_t, rcat,
                  s.astype(jnp.int32), r.astype(jnp.int32),
                  o.astype(jnp.int32))
